# baseline (speedup 1.0000x reference)
import jax
import jax.numpy as jnp
from jax import lax
from jax.experimental import pallas as pl
from jax.experimental.pallas import tpu as pltpu

_VC = 65536

_NT = (((1,), (1,)), ((), ()))


def _pool_body(clus_ref, xt_ref, out_ref, acc_ref, cnt_ref):
    i = pl.program_id(0)

    @pl.when(i == 0)
    def _init():
        acc_ref[...] = jnp.zeros_like(acc_ref)
        cnt_ref[...] = jnp.zeros_like(cnt_ref)

    mask = (clus_ref[...] == 1).astype(jnp.bfloat16)
    xb = xt_ref[...]
    acc_ref[...] += lax.dot_general(
        xb, mask, _NT, preferred_element_type=jnp.float32)
    cnt_ref[...] += lax.dot_general(
        jnp.ones((1, _VC), jnp.bfloat16), mask, _NT,
        preferred_element_type=jnp.float32)

    @pl.when(i == pl.num_programs(0) - 1)
    def _finish():
        out_ref[...] = acc_ref[...] / cnt_ref[...]


def kernel(x, clusters):
    B, V, D = x.shape
    C = clusters.shape[0]
    xt = x[0].astype(jnp.bfloat16).T
    grid = V // _VC
    means_t = pl.pallas_call(
        _pool_body,
        grid=(grid,),
        in_specs=[
            pl.BlockSpec((C, _VC), lambda i: (0, i)),
            pl.BlockSpec((D, _VC), lambda i: (0, i)),
        ],
        out_specs=pl.BlockSpec((D, C), lambda i: (0, 0)),
        out_shape=jax.ShapeDtypeStruct((D, C), jnp.float32),
        scratch_shapes=[
            pltpu.VMEM((D, C), jnp.float32),
            pltpu.VMEM((1, C), jnp.float32),
        ],
    )(clusters, xt)
    return jnp.broadcast_to(means_t.T[None], (B, C, D))


# speedup vs baseline: 1.0416x; 1.0416x over previous
import jax
import jax.numpy as jnp
from jax import lax
from jax.experimental import pallas as pl
from jax.experimental.pallas import tpu as pltpu

_VC = 8192

_NT = (((1,), (1,)), ((), ()))


def _pool_body(clus_ref, xt_ref, out_ref, acc_ref, cnt_ref):
    i = pl.program_id(0)

    @pl.when(i == 0)
    def _init():
        acc_ref[...] = jnp.zeros_like(acc_ref)
        cnt_ref[...] = jnp.zeros_like(cnt_ref)

    mask = (clus_ref[...] == 1).astype(jnp.bfloat16)
    xb = xt_ref[...]
    acc_ref[...] += lax.dot_general(
        xb, mask, _NT, preferred_element_type=jnp.float32)
    cnt_ref[...] += lax.dot_general(
        jnp.ones((1, _VC), jnp.bfloat16), mask, _NT,
        preferred_element_type=jnp.float32)

    @pl.when(i == pl.num_programs(0) - 1)
    def _finish():
        out_ref[...] = acc_ref[...] / cnt_ref[...]


def kernel(x, clusters):
    B, V, D = x.shape
    C = clusters.shape[0]
    xt = x[0].astype(jnp.bfloat16).T
    grid = V // _VC
    means_t = pl.pallas_call(
        _pool_body,
        grid=(grid,),
        in_specs=[
            pl.BlockSpec((C, _VC), lambda i: (0, i)),
            pl.BlockSpec((D, _VC), lambda i: (0, i)),
        ],
        out_specs=pl.BlockSpec((D, C), lambda i: (0, 0)),
        out_shape=jax.ShapeDtypeStruct((D, C), jnp.float32),
        scratch_shapes=[
            pltpu.VMEM((D, C), jnp.float32),
            pltpu.VMEM((1, C), jnp.float32),
        ],
    )(clusters, xt)
    return jnp.broadcast_to(means_t.T[None], (B, C, D))


# speedup vs baseline: 1.1415x; 1.0960x over previous
import jax
import jax.numpy as jnp
from jax import lax
from jax.experimental import pallas as pl
from jax.experimental.pallas import tpu as pltpu

_VC = 16384

_NT = (((1,), (1,)), ((), ()))


def _pool_body(clus_ref, xt_ref, out_ref, acc_ref, cnt_ref):
    i = pl.program_id(0)

    @pl.when(i == 0)
    def _init():
        acc_ref[...] = jnp.zeros_like(acc_ref)
        cnt_ref[...] = jnp.zeros_like(cnt_ref)

    mask = (clus_ref[...] == 1).astype(jnp.bfloat16)
    xb = xt_ref[...]
    acc_ref[...] += lax.dot_general(
        xb, mask, _NT, preferred_element_type=jnp.float32)
    cnt_ref[...] += lax.dot_general(
        jnp.ones((1, _VC), jnp.bfloat16), mask, _NT,
        preferred_element_type=jnp.float32)

    @pl.when(i == pl.num_programs(0) - 1)
    def _finish():
        out_ref[...] = acc_ref[...] / cnt_ref[...]


def kernel(x, clusters):
    B, V, D = x.shape
    C = clusters.shape[0]
    xt = x[0].astype(jnp.bfloat16).T
    grid = V // _VC
    means_t = pl.pallas_call(
        _pool_body,
        grid=(grid,),
        in_specs=[
            pl.BlockSpec((C, _VC), lambda i: (0, i)),
            pl.BlockSpec((D, _VC), lambda i: (0, i)),
        ],
        out_specs=pl.BlockSpec((D, C), lambda i: (0, 0)),
        out_shape=jax.ShapeDtypeStruct((D, C), jnp.float32),
        scratch_shapes=[
            pltpu.VMEM((D, C), jnp.float32),
            pltpu.VMEM((1, C), jnp.float32),
        ],
    )(clusters, xt)
    return jnp.broadcast_to(means_t.T[None], (B, C, D))
